# Initial kernel scaffold; baseline (speedup 1.0000x reference)
import jax
import jax.numpy as jnp
from jax.experimental import pallas as pl


def kernel(x, edge_index, pos_edge_label_index, neg_edge_label_index, W1l, b1l, W1r, W2l, b2l, W2r, Wlin, blin):
    raise NotImplementedError("write your pallas kernel here")


# speedup vs baseline: 5.9384x; 5.9384x over previous
import functools

import jax
import jax.numpy as jnp
from jax import lax
from jax.experimental import pallas as pl
from jax.experimental.pallas import tpu as pltpu
from jax.experimental.pallas import tpu_sc as plsc

N = 10000
E = 320000
D = 128
H = 128
L = 65536

NC = 2
NS = 16
NW = NC * NS
EPW = E // NW
CH = 80
NCH = EPW // CH
RCH = N // CH
LPW = L // NW

_F32 = jnp.float32

_SC_PARAMS = pltpu.CompilerParams(use_tc_tiling_on_sc=False)
_SC_GATHER_PARAMS = pltpu.CompilerParams(use_tc_tiling_on_sc=False,
                                         needs_layout_passes=False)


def _mesh():
    return plsc.VectorSubcoreMesh(core_axis_name="c", subcore_axis_name="s")


def _segsum_deg_body(feat_hbm, src_hbm, dst_hbm, zrow_hbm, zdeg_hbm, ones_hbm,
                     agg_hbm, deg_hbm, acc, dacc, idx_s, idx_d, rows, ones_v,
                     stage, stage_d, sem):
    c = lax.axis_index("c")
    s = lax.axis_index("s")
    w = c * NS + s

    @pl.loop(s, RCH, step=NS)
    def _zero(j):
        pltpu.sync_copy(zrow_hbm.at[pl.ds(j * CH, CH)], stage)
        pltpu.sync_copy(stage, acc.at[pl.ds(j * CH, CH)])
        pltpu.sync_copy(zdeg_hbm.at[pl.ds(j * CH, CH)], stage_d)
        pltpu.sync_copy(stage_d, dacc.at[pl.ds(j * CH, CH)])

    pltpu.sync_copy(ones_hbm, ones_v)
    plsc.subcore_barrier()

    @pl.loop(0, NCH)
    def _chunk(i):
        base = w * EPW + i * CH
        pltpu.sync_copy(src_hbm.at[pl.ds(base, CH)], idx_s)
        pltpu.sync_copy(dst_hbm.at[pl.ds(base, CH)], idx_d)
        pltpu.async_copy(feat_hbm.at[idx_s], rows, sem).wait()
        pltpu.sync_copy(rows, acc.at[idx_d], add=True)
        pltpu.sync_copy(ones_v, dacc.at[idx_d], add=True)

    plsc.subcore_barrier()

    @pl.loop(s, RCH, step=NS)
    def _wb(j):
        pltpu.sync_copy(acc.at[pl.ds(j * CH, CH)], stage)
        pltpu.sync_copy(stage, agg_hbm.at[pl.ds(c * N + j * CH, CH)])
        pltpu.sync_copy(dacc.at[pl.ds(j * CH, CH)], stage_d)
        pltpu.sync_copy(stage_d, deg_hbm.at[pl.ds(c * N + j * CH, CH)])


def _segsum_body(feat_hbm, src_hbm, dst_hbm, zrow_hbm,
                 agg_hbm, acc, idx_s, idx_d, rows, stage, sem):
    c = lax.axis_index("c")
    s = lax.axis_index("s")
    w = c * NS + s

    @pl.loop(s, RCH, step=NS)
    def _zero(j):
        pltpu.sync_copy(zrow_hbm.at[pl.ds(j * CH, CH)], stage)
        pltpu.sync_copy(stage, acc.at[pl.ds(j * CH, CH)])

    plsc.subcore_barrier()

    @pl.loop(0, NCH)
    def _chunk(i):
        base = w * EPW + i * CH
        pltpu.sync_copy(src_hbm.at[pl.ds(base, CH)], idx_s)
        pltpu.sync_copy(dst_hbm.at[pl.ds(base, CH)], idx_d)
        pltpu.async_copy(feat_hbm.at[idx_s], rows, sem).wait()
        pltpu.sync_copy(rows, acc.at[idx_d], add=True)

    plsc.subcore_barrier()

    @pl.loop(s, RCH, step=NS)
    def _wb(j):
        pltpu.sync_copy(acc.at[pl.ds(j * CH, CH)], stage)
        pltpu.sync_copy(stage, agg_hbm.at[pl.ds(c * N + j * CH, CH)])


@functools.lru_cache(maxsize=None)
def _make_segsum_deg():
    return pl.kernel(
        _segsum_deg_body,
        out_type=(jax.ShapeDtypeStruct((NC * N, D), _F32),
                  jax.ShapeDtypeStruct((NC * N,), _F32)),
        mesh=_mesh(),
        compiler_params=_SC_PARAMS,
        scratch_types=[
            pltpu.VMEM_SHARED((N, D), _F32),
            pltpu.VMEM_SHARED((N,), _F32),
            pltpu.VMEM((CH,), jnp.int32),
            pltpu.VMEM((CH,), jnp.int32),
            pltpu.VMEM((CH, D), _F32),
            pltpu.VMEM((CH,), _F32),
            pltpu.VMEM((CH, D), _F32),
            pltpu.VMEM((CH,), _F32),
            pltpu.SemaphoreType.DMA,
        ],
    )


@functools.lru_cache(maxsize=None)
def _make_segsum():
    return pl.kernel(
        _segsum_body,
        out_type=(jax.ShapeDtypeStruct((NC * N, D), _F32),),
        mesh=_mesh(),
        compiler_params=_SC_PARAMS,
        scratch_types=[
            pltpu.VMEM_SHARED((N, D), _F32),
            pltpu.VMEM((CH,), jnp.int32),
            pltpu.VMEM((CH,), jnp.int32),
            pltpu.VMEM((CH, D), _F32),
            pltpu.VMEM((CH, D), _F32),
            pltpu.SemaphoreType.DMA,
        ],
    )


def _sc_scores_body(p_hbm, q_hbm, p0_hbm, p1_hbm, n0_hbm, n1_hbm,
                    pos_hbm, neg_hbm, pv, qv, ia, ib, ov):
    c = lax.axis_index("c")
    s = lax.axis_index("s")
    w = c * NS + s
    base = w * LPW
    pltpu.sync_copy(p_hbm, pv)
    pltpu.sync_copy(q_hbm, qv)
    for i0_hbm, i1_hbm, o_hbm in ((p0_hbm, p1_hbm, pos_hbm),
                                  (n0_hbm, n1_hbm, neg_hbm)):
        pltpu.sync_copy(i0_hbm.at[pl.ds(base, LPW)], ia)
        pltpu.sync_copy(i1_hbm.at[pl.ds(base, LPW)], ib)

        @pl.loop(0, LPW, step=16)
        def _blk(k):
            i0 = ia[pl.ds(k, 16)]
            i1 = ib[pl.ds(k, 16)]
            ov[pl.ds(k, 16)] = (plsc.load_gather(pv, [i0])
                                + plsc.load_gather(qv, [i1]))

        pltpu.sync_copy(ov, o_hbm.at[pl.ds(base, LPW)])


@functools.lru_cache(maxsize=None)
def _make_sc_scores():
    return pl.kernel(
        _sc_scores_body,
        out_type=(jax.ShapeDtypeStruct((L,), _F32),
                  jax.ShapeDtypeStruct((L,), _F32)),
        mesh=_mesh(),
        compiler_params=_SC_GATHER_PARAMS,
        scratch_types=[
            pltpu.VMEM((N,), _F32),
            pltpu.VMEM((N,), _F32),
            pltpu.VMEM((LPW,), jnp.int32),
            pltpu.VMEM((LPW,), jnp.int32),
            pltpu.VMEM((LPW,), _F32),
        ],
    )


_RB = 1000


def _tc1_body(a0, a1, d0, d1, x, w1l_t, w1r_t, b1, h1, dinv):
    deg = d0[...] + d1[...]
    inv = 1.0 / jnp.maximum(deg, 1.0)
    mean = (a0[...] + a1[...]) * inv
    h = (jnp.dot(mean, w1l_t[...], preferred_element_type=_F32) + b1[...]
         + jnp.dot(x[...], w1r_t[...], preferred_element_type=_F32))
    h1[...] = jnp.maximum(h, 0.0)
    dinv[...] = inv


def _tc2_body(a0, a1, dv, h1, w2l_t, w2r_t, b2, wa, wb, bl, p, q):
    mean = (a0[...] + a1[...]) * dv[...]
    h2 = (jnp.dot(mean, w2l_t[...], preferred_element_type=_F32) + b2[...]
          + jnp.dot(h1[...], w2r_t[...], preferred_element_type=_F32))
    p[...] = jnp.dot(h2, wa[...], preferred_element_type=_F32) + bl[...]
    q[...] = jnp.dot(h2, wb[...], preferred_element_type=_F32)


def _row_spec(cols):
    return pl.BlockSpec((_RB, cols), lambda i: (i, 0))


def _full_spec(r, c):
    return pl.BlockSpec((r, c), lambda i: (0, 0))


_tc1 = pl.pallas_call(
    _tc1_body,
    grid=(N // _RB,),
    in_specs=[_row_spec(D), _row_spec(D), _row_spec(1), _row_spec(1),
              _row_spec(D), _full_spec(D, H), _full_spec(D, H),
              _full_spec(1, H)],
    out_specs=[_row_spec(H), _row_spec(1)],
    out_shape=(jax.ShapeDtypeStruct((N, H), _F32),
               jax.ShapeDtypeStruct((N, 1), _F32)),
)

_tc2 = pl.pallas_call(
    _tc2_body,
    grid=(N // _RB,),
    in_specs=[_row_spec(D), _row_spec(D), _row_spec(1), _row_spec(H),
              _full_spec(H, H), _full_spec(H, H), _full_spec(1, H),
              _full_spec(H, 1), _full_spec(H, 1), _full_spec(1, 1)],
    out_specs=[_row_spec(1), _row_spec(1)],
    out_shape=(jax.ShapeDtypeStruct((N, 1), _F32),
               jax.ShapeDtypeStruct((N, 1), _F32)),
)


def kernel(x, edge_index, pos_edge_label_index, neg_edge_label_index,
           W1l, b1l, W1r, W2l, b2l, W2r, Wlin, blin):
    src = edge_index[0]
    dst = edge_index[1]
    zrow = jnp.zeros((N, D), _F32)
    zdeg = jnp.zeros((N,), _F32)
    ones = jnp.ones((CH,), _F32)

    agg1, degp = _make_segsum_deg()(x, src, dst, zrow, zdeg, ones)
    h1, dinv = _tc1(agg1[:N], agg1[N:], degp[:N].reshape(N, 1),
                    degp[N:].reshape(N, 1), x,
                    W1l.T, W1r.T, b1l.reshape(1, H))
    (agg2,) = _make_segsum()(h1, src, dst, zrow)
    p, q = _tc2(agg2[:N], agg2[N:], dinv, h1, W2l.T, W2r.T, b2l.reshape(1, H),
                Wlin[0, :H].reshape(H, 1), Wlin[0, H:].reshape(H, 1),
                blin.reshape(1, 1))
    pos_s, neg_s = _make_sc_scores()(
        p.reshape(N), q.reshape(N),
        pos_edge_label_index[0], pos_edge_label_index[1],
        neg_edge_label_index[0], neg_edge_label_index[1])
    return (pos_s, neg_s)


# speedup vs baseline: 12.1712x; 2.0496x over previous
import functools

import jax
import jax.numpy as jnp
from jax import lax
from jax.experimental import pallas as pl
from jax.experimental.pallas import tpu as pltpu
from jax.experimental.pallas import tpu_sc as plsc

N = 10000
E = 320000
D = 128
H = 128
L = 65536

NC = 2
NS = 16
NW = NC * NS
EPW = E // NW
CH = 80
NCH = EPW // CH
RCH = N // CH
LPW = L // NW

_F32 = jnp.float32

_SC_PARAMS = pltpu.CompilerParams(use_tc_tiling_on_sc=False)
_SC_GATHER_PARAMS = pltpu.CompilerParams(use_tc_tiling_on_sc=False,
                                         needs_layout_passes=False)


def _mesh():
    return plsc.VectorSubcoreMesh(core_axis_name="c", subcore_axis_name="s")


def _edge_pipeline(feat_hbm, dst_hbm, idx_all, acc, w,
                   rows0, rows1, idxd0, idxd1, sg0, sg1, sd0, sd1,
                   deg=None):
    dacc, ones_v = deg if deg is not None else (None, None)

    def fire(off, rows_b, idxd_b, sem_g, sem_d):
        pltpu.async_copy(dst_hbm.at[pl.ds(w * EPW + off * CH, CH)],
                         idxd_b, sem_d)
        pltpu.async_copy(feat_hbm.at[idx_all.at[pl.ds(off * CH, CH)]],
                         rows_b, sem_g)

    def drain(off, rows_b, idxd_b, sem_g, sem_d):
        pltpu.make_async_copy(dst_hbm.at[pl.ds(w * EPW + off * CH, CH)],
                              idxd_b, sem_d).wait()
        pltpu.make_async_copy(feat_hbm.at[idx_all.at[pl.ds(off * CH, CH)]],
                              rows_b, sem_g).wait()
        pltpu.sync_copy(rows_b, acc.at[idxd_b], add=True)
        if dacc is not None:
            pltpu.sync_copy(ones_v, dacc.at[idxd_b], add=True)

    fire(0, rows0, idxd0, sg0, sd0)

    @pl.loop(0, NCH - 1, step=2)
    def _pair(i):
        fire(i + 1, rows1, idxd1, sg1, sd1)
        drain(i, rows0, idxd0, sg0, sd0)
        fire(i + 2, rows0, idxd0, sg0, sd0)
        drain(i + 1, rows1, idxd1, sg1, sd1)

    drain(NCH - 1, rows0, idxd0, sg0, sd0)


def _segsum_deg_body(feat_hbm, src_hbm, dst_hbm, zrow_hbm, zdeg_hbm, ones_hbm,
                     agg_hbm, deg_hbm, acc, dacc, idx_all, idxd0, idxd1,
                     rows0, rows1, ones_v, stage, stage_d,
                     sg0, sg1, sd0, sd1):
    c = lax.axis_index("c")
    s = lax.axis_index("s")
    w = c * NS + s

    pltpu.sync_copy(src_hbm.at[pl.ds(w * EPW, EPW)], idx_all)

    @pl.loop(s, RCH, step=NS)
    def _zero(j):
        pltpu.sync_copy(zrow_hbm.at[pl.ds(j * CH, CH)], stage)
        pltpu.sync_copy(stage, acc.at[pl.ds(j * CH, CH)])
        pltpu.sync_copy(zdeg_hbm.at[pl.ds(j * CH, CH)], stage_d)
        pltpu.sync_copy(stage_d, dacc.at[pl.ds(j * CH, CH)])

    pltpu.sync_copy(ones_hbm, ones_v)
    plsc.subcore_barrier()

    _edge_pipeline(feat_hbm, dst_hbm, idx_all, acc, w,
                   rows0, rows1, idxd0, idxd1, sg0, sg1, sd0, sd1,
                   deg=(dacc, ones_v))

    plsc.subcore_barrier()

    @pl.loop(s, RCH, step=NS)
    def _wb(j):
        pltpu.sync_copy(acc.at[pl.ds(j * CH, CH)], stage)
        pltpu.sync_copy(stage, agg_hbm.at[pl.ds(c * N + j * CH, CH)])
        pltpu.sync_copy(dacc.at[pl.ds(j * CH, CH)], stage_d)
        pltpu.sync_copy(stage_d, deg_hbm.at[pl.ds(c * N + j * CH, CH)])


def _segsum_body(feat_hbm, src_hbm, dst_hbm, zrow_hbm,
                 agg_hbm, acc, idx_all, idxd0, idxd1, rows0, rows1, stage,
                 sg0, sg1, sd0, sd1):
    c = lax.axis_index("c")
    s = lax.axis_index("s")
    w = c * NS + s

    pltpu.sync_copy(src_hbm.at[pl.ds(w * EPW, EPW)], idx_all)

    @pl.loop(s, RCH, step=NS)
    def _zero(j):
        pltpu.sync_copy(zrow_hbm.at[pl.ds(j * CH, CH)], stage)
        pltpu.sync_copy(stage, acc.at[pl.ds(j * CH, CH)])

    plsc.subcore_barrier()

    _edge_pipeline(feat_hbm, dst_hbm, idx_all, acc, w,
                   rows0, rows1, idxd0, idxd1, sg0, sg1, sd0, sd1)

    plsc.subcore_barrier()

    @pl.loop(s, RCH, step=NS)
    def _wb(j):
        pltpu.sync_copy(acc.at[pl.ds(j * CH, CH)], stage)
        pltpu.sync_copy(stage, agg_hbm.at[pl.ds(c * N + j * CH, CH)])


@functools.lru_cache(maxsize=None)
def _make_segsum_deg():
    return pl.kernel(
        _segsum_deg_body,
        out_type=(jax.ShapeDtypeStruct((NC * N, D), _F32),
                  jax.ShapeDtypeStruct((NC * N,), _F32)),
        mesh=_mesh(),
        compiler_params=_SC_PARAMS,
        scratch_types=[
            pltpu.VMEM_SHARED((N, D), _F32),
            pltpu.VMEM_SHARED((N,), _F32),
            pltpu.VMEM((EPW,), jnp.int32),
            pltpu.VMEM((CH,), jnp.int32),
            pltpu.VMEM((CH,), jnp.int32),
            pltpu.VMEM((CH, D), _F32),
            pltpu.VMEM((CH, D), _F32),
            pltpu.VMEM((CH,), _F32),
            pltpu.VMEM((CH, D), _F32),
            pltpu.VMEM((CH,), _F32),
            pltpu.SemaphoreType.DMA,
            pltpu.SemaphoreType.DMA,
            pltpu.SemaphoreType.DMA,
            pltpu.SemaphoreType.DMA,
        ],
    )


@functools.lru_cache(maxsize=None)
def _make_segsum():
    return pl.kernel(
        _segsum_body,
        out_type=(jax.ShapeDtypeStruct((NC * N, D), _F32),),
        mesh=_mesh(),
        compiler_params=_SC_PARAMS,
        scratch_types=[
            pltpu.VMEM_SHARED((N, D), _F32),
            pltpu.VMEM((EPW,), jnp.int32),
            pltpu.VMEM((CH,), jnp.int32),
            pltpu.VMEM((CH,), jnp.int32),
            pltpu.VMEM((CH, D), _F32),
            pltpu.VMEM((CH, D), _F32),
            pltpu.VMEM((CH, D), _F32),
            pltpu.SemaphoreType.DMA,
            pltpu.SemaphoreType.DMA,
            pltpu.SemaphoreType.DMA,
            pltpu.SemaphoreType.DMA,
        ],
    )


def _sc_scores_body(p_hbm, q_hbm, p0_hbm, p1_hbm, n0_hbm, n1_hbm,
                    pos_hbm, neg_hbm, pv, qv, ia, ib, ov):
    c = lax.axis_index("c")
    s = lax.axis_index("s")
    w = c * NS + s
    base = w * LPW
    pltpu.sync_copy(p_hbm, pv)
    pltpu.sync_copy(q_hbm, qv)
    for i0_hbm, i1_hbm, o_hbm in ((p0_hbm, p1_hbm, pos_hbm),
                                  (n0_hbm, n1_hbm, neg_hbm)):
        pltpu.sync_copy(i0_hbm.at[pl.ds(base, LPW)], ia)
        pltpu.sync_copy(i1_hbm.at[pl.ds(base, LPW)], ib)

        @pl.loop(0, LPW, step=16)
        def _blk(k):
            i0 = ia[pl.ds(k, 16)]
            i1 = ib[pl.ds(k, 16)]
            ov[pl.ds(k, 16)] = (plsc.load_gather(pv, [i0])
                                + plsc.load_gather(qv, [i1]))

        pltpu.sync_copy(ov, o_hbm.at[pl.ds(base, LPW)])


@functools.lru_cache(maxsize=None)
def _make_sc_scores():
    return pl.kernel(
        _sc_scores_body,
        out_type=(jax.ShapeDtypeStruct((L,), _F32),
                  jax.ShapeDtypeStruct((L,), _F32)),
        mesh=_mesh(),
        compiler_params=_SC_GATHER_PARAMS,
        scratch_types=[
            pltpu.VMEM((N,), _F32),
            pltpu.VMEM((N,), _F32),
            pltpu.VMEM((LPW,), jnp.int32),
            pltpu.VMEM((LPW,), jnp.int32),
            pltpu.VMEM((LPW,), _F32),
        ],
    )


_RB = 1000


def _tc1_body(a0, a1, d0, d1, x, w1l_t, w1r_t, b1, h1, dinv):
    deg = d0[...] + d1[...]
    inv = 1.0 / jnp.maximum(deg, 1.0)
    mean = (a0[...] + a1[...]) * inv
    h = (jnp.dot(mean, w1l_t[...], preferred_element_type=_F32) + b1[...]
         + jnp.dot(x[...], w1r_t[...], preferred_element_type=_F32))
    h1[...] = jnp.maximum(h, 0.0)
    dinv[...] = inv


def _tc2_body(a0, a1, dv, h1, w2l_t, w2r_t, b2, wa, wb, bl, p, q):
    mean = (a0[...] + a1[...]) * dv[...]
    h2 = (jnp.dot(mean, w2l_t[...], preferred_element_type=_F32) + b2[...]
          + jnp.dot(h1[...], w2r_t[...], preferred_element_type=_F32))
    p[...] = jnp.dot(h2, wa[...], preferred_element_type=_F32) + bl[...]
    q[...] = jnp.dot(h2, wb[...], preferred_element_type=_F32)


def _row_spec(cols):
    return pl.BlockSpec((_RB, cols), lambda i: (i, 0))


def _full_spec(r, c):
    return pl.BlockSpec((r, c), lambda i: (0, 0))


_tc1 = pl.pallas_call(
    _tc1_body,
    grid=(N // _RB,),
    in_specs=[_row_spec(D), _row_spec(D), _row_spec(1), _row_spec(1),
              _row_spec(D), _full_spec(D, H), _full_spec(D, H),
              _full_spec(1, H)],
    out_specs=[_row_spec(H), _row_spec(1)],
    out_shape=(jax.ShapeDtypeStruct((N, H), _F32),
               jax.ShapeDtypeStruct((N, 1), _F32)),
)

_tc2 = pl.pallas_call(
    _tc2_body,
    grid=(N // _RB,),
    in_specs=[_row_spec(D), _row_spec(D), _row_spec(1), _row_spec(H),
              _full_spec(H, H), _full_spec(H, H), _full_spec(1, H),
              _full_spec(H, 1), _full_spec(H, 1), _full_spec(1, 1)],
    out_specs=[_row_spec(1), _row_spec(1)],
    out_shape=(jax.ShapeDtypeStruct((N, 1), _F32),
               jax.ShapeDtypeStruct((N, 1), _F32)),
)


def kernel(x, edge_index, pos_edge_label_index, neg_edge_label_index,
           W1l, b1l, W1r, W2l, b2l, W2r, Wlin, blin):
    src = edge_index[0]
    dst = edge_index[1]
    zrow = jnp.zeros((N, D), _F32)
    zdeg = jnp.zeros((N,), _F32)
    ones = jnp.ones((CH,), _F32)

    agg1, degp = _make_segsum_deg()(x, src, dst, zrow, zdeg, ones)
    h1, dinv = _tc1(agg1[:N], agg1[N:], degp[:N].reshape(N, 1),
                    degp[N:].reshape(N, 1), x,
                    W1l.T, W1r.T, b1l.reshape(1, H))
    (agg2,) = _make_segsum()(h1, src, dst, zrow)
    p, q = _tc2(agg2[:N], agg2[N:], dinv, h1, W2l.T, W2r.T, b2l.reshape(1, H),
                Wlin[0, :H].reshape(H, 1), Wlin[0, H:].reshape(H, 1),
                blin.reshape(1, 1))
    pos_s, neg_s = _make_sc_scores()(
        p.reshape(N), q.reshape(N),
        pos_edge_label_index[0], pos_edge_label_index[1],
        neg_edge_label_index[0], neg_edge_label_index[1])
    return (pos_s, neg_s)


# speedup vs baseline: 12.6958x; 1.0431x over previous
import functools

import jax
import jax.numpy as jnp
from jax import lax
from jax.experimental import pallas as pl
from jax.experimental.pallas import tpu as pltpu
from jax.experimental.pallas import tpu_sc as plsc

N = 10000
E = 320000
D = 128
H = 128
L = 65536

NC = 2
NS = 16
NW = NC * NS
EPW = E // NW
CH = 80
NCH = EPW // CH
RCH = N // CH
LPW = L // NW

_F32 = jnp.float32

_SC_PARAMS = pltpu.CompilerParams(use_tc_tiling_on_sc=False)
_SC_GATHER_PARAMS = pltpu.CompilerParams(use_tc_tiling_on_sc=False,
                                         needs_layout_passes=False)


def _mesh():
    return plsc.VectorSubcoreMesh(core_axis_name="c", subcore_axis_name="s")


def _edge_pipeline(feat_hbm, dst_hbm, idx_all, acc, w,
                   rows0, rows1, idxd0, idxd1, sg0, sg1, sd0, sd1,
                   deg=None):
    dacc, ones_v = deg if deg is not None else (None, None)

    def fire(off, rows_b, idxd_b, sem_g, sem_d):
        pltpu.async_copy(dst_hbm.at[pl.ds(w * EPW + off * CH, CH)],
                         idxd_b, sem_d)
        pltpu.async_copy(feat_hbm.at[idx_all.at[pl.ds(off * CH, CH)]],
                         rows_b, sem_g)

    def drain(off, rows_b, idxd_b, sem_g, sem_d):
        pltpu.make_async_copy(dst_hbm.at[pl.ds(w * EPW + off * CH, CH)],
                              idxd_b, sem_d).wait()
        pltpu.make_async_copy(feat_hbm.at[idx_all.at[pl.ds(off * CH, CH)]],
                              rows_b, sem_g).wait()
        pltpu.sync_copy(rows_b, acc.at[idxd_b], add=True)
        if dacc is not None:
            pltpu.sync_copy(ones_v, dacc.at[idxd_b], add=True)

    fire(0, rows0, idxd0, sg0, sd0)

    @pl.loop(0, NCH - 1, step=2)
    def _pair(i):
        fire(i + 1, rows1, idxd1, sg1, sd1)
        drain(i, rows0, idxd0, sg0, sd0)
        fire(i + 2, rows0, idxd0, sg0, sd0)
        drain(i + 1, rows1, idxd1, sg1, sd1)

    drain(NCH - 1, rows0, idxd0, sg0, sd0)


def _segsum_deg_body(feat_hbm, src_hbm, dst_hbm, zrow_hbm, zdeg_hbm, ones_hbm,
                     agg_hbm, deg_hbm, acc, dacc, idx_all, idxd0, idxd1,
                     rows0, rows1, ones_v, stage, stage_d,
                     sg0, sg1, sd0, sd1):
    c = lax.axis_index("c")
    s = lax.axis_index("s")
    w = c * NS + s

    pltpu.sync_copy(src_hbm.at[pl.ds(w * EPW, EPW)], idx_all)
    pltpu.sync_copy(zrow_hbm, stage)
    pltpu.sync_copy(zdeg_hbm, stage_d)

    @pl.loop(s, RCH, step=NS)
    def _zero(j):
        pltpu.sync_copy(stage, acc.at[pl.ds(j * CH, CH)])
        pltpu.sync_copy(stage_d, dacc.at[pl.ds(j * CH, CH)])

    pltpu.sync_copy(ones_hbm, ones_v)
    plsc.subcore_barrier()

    _edge_pipeline(feat_hbm, dst_hbm, idx_all, acc, w,
                   rows0, rows1, idxd0, idxd1, sg0, sg1, sd0, sd1,
                   deg=(dacc, ones_v))

    plsc.subcore_barrier()

    @pl.loop(s, RCH, step=NS)
    def _wb(j):
        pltpu.sync_copy(acc.at[pl.ds(j * CH, CH)],
                        agg_hbm.at[pl.ds(c * N + j * CH, CH)])
        pltpu.sync_copy(dacc.at[pl.ds(j * CH, CH)],
                        deg_hbm.at[pl.ds(c * N + j * CH, CH)])


def _segsum_body(feat_hbm, src_hbm, dst_hbm, zrow_hbm,
                 agg_hbm, acc, idx_all, idxd0, idxd1, rows0, rows1, stage,
                 sg0, sg1, sd0, sd1):
    c = lax.axis_index("c")
    s = lax.axis_index("s")
    w = c * NS + s

    pltpu.sync_copy(src_hbm.at[pl.ds(w * EPW, EPW)], idx_all)
    pltpu.sync_copy(zrow_hbm, stage)

    @pl.loop(s, RCH, step=NS)
    def _zero(j):
        pltpu.sync_copy(stage, acc.at[pl.ds(j * CH, CH)])

    plsc.subcore_barrier()

    _edge_pipeline(feat_hbm, dst_hbm, idx_all, acc, w,
                   rows0, rows1, idxd0, idxd1, sg0, sg1, sd0, sd1)

    plsc.subcore_barrier()

    @pl.loop(s, RCH, step=NS)
    def _wb(j):
        pltpu.sync_copy(acc.at[pl.ds(j * CH, CH)],
                        agg_hbm.at[pl.ds(c * N + j * CH, CH)])


@functools.lru_cache(maxsize=None)
def _make_segsum_deg():
    return pl.kernel(
        _segsum_deg_body,
        out_type=(jax.ShapeDtypeStruct((NC * N, D), _F32),
                  jax.ShapeDtypeStruct((NC * N,), _F32)),
        mesh=_mesh(),
        compiler_params=_SC_PARAMS,
        scratch_types=[
            pltpu.VMEM_SHARED((N, D), _F32),
            pltpu.VMEM_SHARED((N,), _F32),
            pltpu.VMEM((EPW,), jnp.int32),
            pltpu.VMEM((CH,), jnp.int32),
            pltpu.VMEM((CH,), jnp.int32),
            pltpu.VMEM((CH, D), _F32),
            pltpu.VMEM((CH, D), _F32),
            pltpu.VMEM((CH,), _F32),
            pltpu.VMEM((CH, D), _F32),
            pltpu.VMEM((CH,), _F32),
            pltpu.SemaphoreType.DMA,
            pltpu.SemaphoreType.DMA,
            pltpu.SemaphoreType.DMA,
            pltpu.SemaphoreType.DMA,
        ],
    )


@functools.lru_cache(maxsize=None)
def _make_segsum():
    return pl.kernel(
        _segsum_body,
        out_type=(jax.ShapeDtypeStruct((NC * N, D), _F32),),
        mesh=_mesh(),
        compiler_params=_SC_PARAMS,
        scratch_types=[
            pltpu.VMEM_SHARED((N, D), _F32),
            pltpu.VMEM((EPW,), jnp.int32),
            pltpu.VMEM((CH,), jnp.int32),
            pltpu.VMEM((CH,), jnp.int32),
            pltpu.VMEM((CH, D), _F32),
            pltpu.VMEM((CH, D), _F32),
            pltpu.VMEM((CH, D), _F32),
            pltpu.SemaphoreType.DMA,
            pltpu.SemaphoreType.DMA,
            pltpu.SemaphoreType.DMA,
            pltpu.SemaphoreType.DMA,
        ],
    )


def _sc_scores_body(p_hbm, q_hbm, p0_hbm, p1_hbm, n0_hbm, n1_hbm,
                    pos_hbm, neg_hbm, pv, qv, ia, ib, ov):
    c = lax.axis_index("c")
    s = lax.axis_index("s")
    w = c * NS + s
    base = w * LPW
    pltpu.sync_copy(p_hbm, pv)
    pltpu.sync_copy(q_hbm, qv)
    for i0_hbm, i1_hbm, o_hbm in ((p0_hbm, p1_hbm, pos_hbm),
                                  (n0_hbm, n1_hbm, neg_hbm)):
        pltpu.sync_copy(i0_hbm.at[pl.ds(base, LPW)], ia)
        pltpu.sync_copy(i1_hbm.at[pl.ds(base, LPW)], ib)

        @pl.loop(0, LPW, step=16)
        def _blk(k):
            i0 = ia[pl.ds(k, 16)]
            i1 = ib[pl.ds(k, 16)]
            ov[pl.ds(k, 16)] = (plsc.load_gather(pv, [i0])
                                + plsc.load_gather(qv, [i1]))

        pltpu.sync_copy(ov, o_hbm.at[pl.ds(base, LPW)])


@functools.lru_cache(maxsize=None)
def _make_sc_scores():
    return pl.kernel(
        _sc_scores_body,
        out_type=(jax.ShapeDtypeStruct((L,), _F32),
                  jax.ShapeDtypeStruct((L,), _F32)),
        mesh=_mesh(),
        compiler_params=_SC_GATHER_PARAMS,
        scratch_types=[
            pltpu.VMEM((N,), _F32),
            pltpu.VMEM((N,), _F32),
            pltpu.VMEM((LPW,), jnp.int32),
            pltpu.VMEM((LPW,), jnp.int32),
            pltpu.VMEM((LPW,), _F32),
        ],
    )


_RB = 1000


def _tc1_body(a0, a1, d0, d1, x, w1l_t, w1r_t, b1, h1, dinv):
    deg = d0[...] + d1[...]
    inv = 1.0 / jnp.maximum(deg, 1.0)
    mean = (a0[...] + a1[...]) * inv
    h = (jnp.dot(mean, w1l_t[...], preferred_element_type=_F32) + b1[...]
         + jnp.dot(x[...], w1r_t[...], preferred_element_type=_F32))
    h1[...] = jnp.maximum(h, 0.0)
    dinv[...] = inv


def _tc2_body(a0, a1, dv, h1, w2l_t, w2r_t, b2, wa, wb, bl, p, q):
    mean = (a0[...] + a1[...]) * dv[...]
    h2 = (jnp.dot(mean, w2l_t[...], preferred_element_type=_F32) + b2[...]
          + jnp.dot(h1[...], w2r_t[...], preferred_element_type=_F32))
    p[...] = jnp.dot(h2, wa[...], preferred_element_type=_F32) + bl[...]
    q[...] = jnp.dot(h2, wb[...], preferred_element_type=_F32)


def _row_spec(cols):
    return pl.BlockSpec((_RB, cols), lambda i: (i, 0))


def _full_spec(r, c):
    return pl.BlockSpec((r, c), lambda i: (0, 0))


_tc1 = pl.pallas_call(
    _tc1_body,
    grid=(N // _RB,),
    in_specs=[_row_spec(D), _row_spec(D), _row_spec(1), _row_spec(1),
              _row_spec(D), _full_spec(D, H), _full_spec(D, H),
              _full_spec(1, H)],
    out_specs=[_row_spec(H), _row_spec(1)],
    out_shape=(jax.ShapeDtypeStruct((N, H), _F32),
               jax.ShapeDtypeStruct((N, 1), _F32)),
)

_tc2 = pl.pallas_call(
    _tc2_body,
    grid=(N // _RB,),
    in_specs=[_row_spec(D), _row_spec(D), _row_spec(1), _row_spec(H),
              _full_spec(H, H), _full_spec(H, H), _full_spec(1, H),
              _full_spec(H, 1), _full_spec(H, 1), _full_spec(1, 1)],
    out_specs=[_row_spec(1), _row_spec(1)],
    out_shape=(jax.ShapeDtypeStruct((N, 1), _F32),
               jax.ShapeDtypeStruct((N, 1), _F32)),
)


def kernel(x, edge_index, pos_edge_label_index, neg_edge_label_index,
           W1l, b1l, W1r, W2l, b2l, W2r, Wlin, blin):
    src = edge_index[0]
    dst = edge_index[1]
    zrow = jnp.zeros((CH, D), _F32)
    zdeg = jnp.zeros((CH,), _F32)
    ones = jnp.ones((CH,), _F32)

    agg1, degp = _make_segsum_deg()(x, src, dst, zrow, zdeg, ones)
    h1, dinv = _tc1(agg1[:N], agg1[N:], degp[:N].reshape(N, 1),
                    degp[N:].reshape(N, 1), x,
                    W1l.T, W1r.T, b1l.reshape(1, H))
    (agg2,) = _make_segsum()(h1, src, dst, zrow)
    p, q = _tc2(agg2[:N], agg2[N:], dinv, h1, W2l.T, W2r.T, b2l.reshape(1, H),
                Wlin[0, :H].reshape(H, 1), Wlin[0, H:].reshape(H, 1),
                blin.reshape(1, 1))
    pos_s, neg_s = _make_sc_scores()(
        p.reshape(N), q.reshape(N),
        pos_edge_label_index[0], pos_edge_label_index[1],
        neg_edge_label_index[0], neg_edge_label_index[1])
    return (pos_s, neg_s)
